# baseline (device time: 14729 ns/iter reference)
import jax
import jax.numpy as jnp
from jax import lax
from jax.experimental import pallas as pl
from jax.experimental.pallas import tpu as pltpu

N_DEV = 4
B, SQ_PER, SKV_PER, HQ, DH = 2, 128, 128, 4, 64
D_MODEL = 512
D_QK = HQ * DH
BLK = 64


def kernel(x, Wq, K_ext, V_ext, Wo):
    def body(x_hbm, wq_hbm, kt_hbm, vt_hbm, wo_hbm, out_hbm,
             xs, wqs, kts, vts, wos, os_ref, kv_ref,
             in_sems, out_sems, send_sems, recv_sems):
        my = lax.axis_index("i")
        partner = (my + 2) % N_DEV

        barrier_sem = pltpu.get_barrier_semaphore()
        pl.semaphore_signal(
            barrier_sem, inc=1,
            device_id=(partner,), device_id_type=pl.DeviceIdType.MESH,
        )

        dmas = []
        for i, (src, dst) in enumerate(
                [(kt_hbm, kts), (vt_hbm, vts), (x_hbm, xs),
                 (wq_hbm, wqs), (wo_hbm, wos)]):
            d = pltpu.make_async_copy(src, dst, in_sems.at[i])
            d.start()
            dmas.append(d)

        def send(b):
            r = pltpu.make_async_remote_copy(
                src_ref=kv_ref.at[0, b], dst_ref=kv_ref.at[1, b],
                send_sem=send_sems.at[b], recv_sem=recv_sems.at[b],
                device_id=(partner,), device_id_type=pl.DeviceIdType.MESH,
            )
            r.start()
            return r

        dmas[0].wait()
        dmas[1].wait()
        rdmas = []
        for b in range(B):
            kv_ref[0, b, :D_QK, :] = (
                kts[b].astype(jnp.bfloat16).reshape(D_QK, SKV_PER))
            kv_ref[0, b, D_QK:, :] = (
                vts[b].astype(jnp.bfloat16).reshape(D_QK, SKV_PER))
            if b == 0:
                pl.semaphore_wait(barrier_sem, 1)
            rdmas.append(send(b))

        dmas[2].wait()
        dmas[3].wait()
        wq = wqs[...].astype(jnp.bfloat16)
        xx = xs[...].astype(jnp.bfloat16).reshape(B * SQ_PER, D_MODEL)
        q2 = (jnp.dot(xx, wq, preferred_element_type=jnp.float32)
              * 0.125).astype(jnp.bfloat16)
        dmas[4].wait()
        wo = wos[...].astype(jnp.bfloat16)

        out_dmas = []
        for b in range(B):
            rdmas[b].wait_recv()
            ctx_rows = []
            for t in range(2):
                r0 = b * SQ_PER + t * BLK
                heads = []
                for h in range(HQ):
                    q = q2[r0:r0 + BLK, h * DH:(h + 1) * DH]
                    krows = pl.ds(h * DH, DH)
                    vrows = pl.ds(D_QK + h * DH, DH)
                    scols = pl.ds(t * BLK, BLK)
                    s_l = jnp.dot(q, kv_ref[0, b, krows, scols],
                                  preferred_element_type=jnp.float32)
                    s_r = jnp.dot(q, kv_ref[1, b, krows, scols],
                                  preferred_element_type=jnp.float32)
                    w_l = jnp.exp(s_l)
                    w_r = jnp.exp(s_r)
                    wsum = (jnp.sum(w_l, axis=-1, keepdims=True)
                            + jnp.sum(w_r, axis=-1, keepdims=True))
                    dn = (((1,), (1,)), ((), ()))
                    ctx = (
                        lax.dot_general(
                            w_l.astype(jnp.bfloat16),
                            kv_ref[0, b, vrows, scols],
                            dimension_numbers=dn,
                            preferred_element_type=jnp.float32)
                        + lax.dot_general(
                            w_r.astype(jnp.bfloat16),
                            kv_ref[1, b, vrows, scols],
                            dimension_numbers=dn,
                            preferred_element_type=jnp.float32)
                    )
                    heads.append((ctx * (1.0 / wsum)).astype(jnp.bfloat16))
                ctx_rows.append(jnp.concatenate(heads, axis=1))
            ctx_b = jnp.concatenate(ctx_rows, axis=0)
            os_ref[b] = jnp.dot(ctx_b, wo, preferred_element_type=jnp.float32)
            d = pltpu.make_async_copy(os_ref.at[b], out_hbm.at[b],
                                      out_sems.at[b])
            d.start()
            out_dmas.append(d)

        for d in out_dmas:
            d.wait()
        for b in range(B):
            rdmas[b].wait_send()

    K_t = jnp.transpose(K_ext, (0, 2, 3, 1))
    V_t = jnp.transpose(V_ext, (0, 2, 3, 1))

    return pl.pallas_call(
        body,
        out_shape=jax.ShapeDtypeStruct((B, SQ_PER, D_MODEL), jnp.float32),
        in_specs=[pl.BlockSpec(memory_space=pl.ANY)] * 5,
        out_specs=pl.BlockSpec(memory_space=pl.ANY),
        scratch_shapes=[
            pltpu.VMEM((B, SQ_PER, D_MODEL), jnp.float32),
            pltpu.VMEM((D_MODEL, D_QK), jnp.float32),
            pltpu.VMEM((B, HQ, DH, SKV_PER), jnp.float32),
            pltpu.VMEM((B, HQ, DH, SKV_PER), jnp.float32),
            pltpu.VMEM((D_QK, D_MODEL), jnp.float32),
            pltpu.VMEM((B, SQ_PER, D_MODEL), jnp.float32),
            pltpu.VMEM((2, B, 2 * D_QK, SKV_PER), jnp.bfloat16),
            pltpu.SemaphoreType.DMA((5,)),
            pltpu.SemaphoreType.DMA((B,)),
            pltpu.SemaphoreType.DMA((B,)),
            pltpu.SemaphoreType.DMA((B,)),
        ],
        compiler_params=pltpu.CompilerParams(collective_id=0),
    )(x, Wq, K_t, V_t, Wo)
